# baseline (speedup 1.0000x reference)
import jax
import jax.numpy as jnp
from jax import lax
from jax.experimental import pallas as pl
from jax.experimental.pallas import tpu as pltpu
from jax.experimental.pallas import tpu_sc as plsc

D = 128
E = 320000
NC = 2
NS = 16
NW = NC * NS
EPW = E // NW
C = 80
NCHUNK = EPW // C
NBUF = 4
UNROLL = 8


def _edge_dot_body(tab_hbm, cidx_hbm, out_hbm, cidx_v, out_v, *bufs_and_sems):
    rows = bufs_and_sems[0:NBUF]
    sems = bufs_and_sems[NBUF:]
    wid = lax.axis_index("s") * NC + lax.axis_index("c")
    wbase = wid * EPW

    pltpu.sync_copy(cidx_hbm.at[pl.ds(wbase * 2, EPW * 2)], cidx_v)

    def fire(ci, b):
        pltpu.async_copy(
            tab_hbm.at[cidx_v.at[pl.ds(ci * 2 * C, 2 * C)]], rows[b], sems[b])

    def wait(b):
        pltpu.make_async_copy(
            tab_hbm.at[pl.ds(0, 2 * C)], rows[b], sems[b]).wait()

    def compute(ci, b):
        rb = rows[b]
        lane = lax.iota(jnp.int32, 16)
        for g in range(C // 16):
            srow = lane + g * 16
            trow = srow + C
            zero = jnp.zeros((16,), jnp.float32)

            def d_blk(k, carry):
                acc0, acc1 = carry
                base = k * UNROLL
                for j in range(UNROLL):
                    col = (jnp.full((16,), base + j, jnp.int32) + lane) & (D - 1)
                    s = plsc.load_gather(rb, [srow, col])
                    t = plsc.load_gather(rb, [trow, col])
                    if j % 2 == 0:
                        acc0 = acc0 + s * t
                    else:
                        acc1 = acc1 + s * t
                return acc0, acc1

            acc0, acc1 = lax.fori_loop(0, D // UNROLL, d_blk, (zero, zero))
            out_v[pl.ds(ci * C + g * 16, 16)] = acc0 + acc1

    for b in range(NBUF):
        fire(b, b)

    def loop_body(i, carry):
        for b in range(NBUF):
            ci = i * NBUF + b

            @pl.when(ci < NCHUNK)
            def _():
                wait(b)
                compute(ci, b)

                @pl.when(ci + NBUF < NCHUNK)
                def _():
                    fire(ci + NBUF, b)

        return carry

    lax.fori_loop(0, (NCHUNK + NBUF - 1) // NBUF, loop_body, 0)
    pltpu.sync_copy(out_v, out_hbm.at[pl.ds(wbase, EPW)])


def kernel(node_src_feats, node_tgt_feats, edge_ids):
    eids = edge_ids.astype(jnp.int32)
    nn = node_src_feats.shape[0]
    table = jnp.concatenate([node_src_feats, node_tgt_feats], axis=0)
    sids = eids[0].reshape(NW, NCHUNK, C)
    tids = eids[1].reshape(NW, NCHUNK, C) + nn
    cidx = jnp.stack([sids, tids], axis=2).reshape(-1)
    mesh = plsc.VectorSubcoreMesh(core_axis_name="c", subcore_axis_name="s")
    fn = pl.kernel(
        _edge_dot_body,
        out_type=jax.ShapeDtypeStruct((E,), jnp.float32),
        mesh=mesh,
        scratch_types=[
            pltpu.VMEM((EPW * 2,), jnp.int32),
            pltpu.VMEM((EPW,), jnp.float32),
        ] + [pltpu.VMEM((2 * C, D), jnp.float32) for _ in range(NBUF)]
          + [pltpu.SemaphoreType.DMA for _ in range(NBUF)],
        compiler_params=pltpu.CompilerParams(
            needs_layout_passes=False, use_tc_tiling_on_sc=False),
    )
    return fn(table, cidx)


# speedup vs baseline: 1.1940x; 1.1940x over previous
import jax
import jax.numpy as jnp
from jax import lax
from jax.experimental import pallas as pl
from jax.experimental.pallas import tpu as pltpu
from jax.experimental.pallas import tpu_sc as plsc

D = 128
E = 320000
NC = 2
NS = 16
NW = NC * NS
EPW = E // NW
C = 80
NCHUNK = EPW // C
NBUF = 4
SPLIT = 2
CS = C // SPLIT
UNROLL = 8


def _edge_dot_body(src_hbm, tgt_hbm, sid_hbm, tid_hbm, out_hbm,
                   sidx_v, tidx_v, out_v, *bufs_and_sems):
    srows = bufs_and_sems[0:NBUF]
    trows = bufs_and_sems[NBUF:2 * NBUF]
    sems = bufs_and_sems[2 * NBUF:]
    wid = lax.axis_index("s") * NC + lax.axis_index("c")
    wbase = wid * EPW

    pltpu.sync_copy(sid_hbm.at[pl.ds(wbase, EPW)], sidx_v)
    pltpu.sync_copy(tid_hbm.at[pl.ds(wbase, EPW)], tidx_v)

    def fire(ci, b):
        for h in range(SPLIT):
            pltpu.async_copy(
                src_hbm.at[sidx_v.at[pl.ds(ci * C + h * CS, CS)]],
                srows[b].at[pl.ds(h * CS, CS)], sems[2 * b])
            pltpu.async_copy(
                tgt_hbm.at[tidx_v.at[pl.ds(ci * C + h * CS, CS)]],
                trows[b].at[pl.ds(h * CS, CS)], sems[2 * b + 1])

    def wait(b):
        pltpu.make_async_copy(
            src_hbm.at[pl.ds(0, C)], srows[b], sems[2 * b]).wait()
        pltpu.make_async_copy(
            tgt_hbm.at[pl.ds(0, C)], trows[b], sems[2 * b + 1]).wait()

    def compute(ci, b):
        sb = srows[b]
        tb = trows[b]
        lane = lax.iota(jnp.int32, 16)
        for g in range(C // 16):
            rows = lane + g * 16
            zero = jnp.zeros((16,), jnp.float32)

            def d_blk(k, carry):
                acc0, acc1 = carry
                base = k * UNROLL
                for j in range(UNROLL):
                    col = (jnp.full((16,), base + j, jnp.int32) + lane) & (D - 1)
                    s = plsc.load_gather(sb, [rows, col])
                    t = plsc.load_gather(tb, [rows, col])
                    if j % 2 == 0:
                        acc0 = acc0 + s * t
                    else:
                        acc1 = acc1 + s * t
                return acc0, acc1

            acc0, acc1 = lax.fori_loop(0, D // UNROLL, d_blk, (zero, zero))
            out_v[pl.ds(ci * C + g * 16, 16)] = acc0 + acc1

    for b in range(NBUF):
        fire(b, b)

    def loop_body(i, carry):
        for b in range(NBUF):
            ci = i * NBUF + b

            @pl.when(ci < NCHUNK)
            def _():
                wait(b)
                compute(ci, b)

                @pl.when(ci + NBUF < NCHUNK)
                def _():
                    fire(ci + NBUF, b)

        return carry

    lax.fori_loop(0, (NCHUNK + NBUF - 1) // NBUF, loop_body, 0)
    pltpu.sync_copy(out_v, out_hbm.at[pl.ds(wbase, EPW)])


def kernel(node_src_feats, node_tgt_feats, edge_ids):
    eids = edge_ids.astype(jnp.int32)
    sids = eids[0]
    tids = eids[1]
    mesh = plsc.VectorSubcoreMesh(core_axis_name="c", subcore_axis_name="s")
    fn = pl.kernel(
        _edge_dot_body,
        out_type=jax.ShapeDtypeStruct((E,), jnp.float32),
        mesh=mesh,
        scratch_types=[
            pltpu.VMEM((EPW,), jnp.int32),
            pltpu.VMEM((EPW,), jnp.int32),
            pltpu.VMEM((EPW,), jnp.float32),
        ] + [pltpu.VMEM((C, D), jnp.float32) for _ in range(2 * NBUF)]
          + [pltpu.SemaphoreType.DMA for _ in range(2 * NBUF)],
        compiler_params=pltpu.CompilerParams(
            needs_layout_passes=False, use_tc_tiling_on_sc=False),
    )
    return fn(node_src_feats, node_tgt_feats, sids, tids)


# speedup vs baseline: 1.2775x; 1.0699x over previous
import jax
import jax.numpy as jnp
from jax import lax
from jax.experimental import pallas as pl
from jax.experimental.pallas import tpu as pltpu
from jax.experimental.pallas import tpu_sc as plsc

D = 128
E = 320000
NC = 2
NS = 16
NW = NC * NS
EPW = E // NW
C = 80
NCHUNK = EPW // C
NBUF = 4
SPLIT = 2
CS = C // SPLIT
UNROLL = 8


def _edge_dot_body(src_hbm, tgt_hbm, ids_hbm, out_hbm,
                   sidx_v, tidx_v, out_v, *bufs_and_sems):
    srows = bufs_and_sems[0:NBUF]
    trows = bufs_and_sems[NBUF:2 * NBUF]
    sems = bufs_and_sems[2 * NBUF:]
    wid = lax.axis_index("s") * NC + lax.axis_index("c")
    wbase = wid * EPW

    pltpu.sync_copy(ids_hbm.at[0, pl.ds(wbase, EPW)], sidx_v)
    pltpu.sync_copy(ids_hbm.at[1, pl.ds(wbase, EPW)], tidx_v)

    def fire(ci, b):
        for h in range(SPLIT):
            pltpu.async_copy(
                src_hbm.at[sidx_v.at[pl.ds(ci * C + h * CS, CS)]],
                srows[b].at[pl.ds(h * CS, CS)], sems[2 * b])
            pltpu.async_copy(
                tgt_hbm.at[tidx_v.at[pl.ds(ci * C + h * CS, CS)]],
                trows[b].at[pl.ds(h * CS, CS)], sems[2 * b + 1])

    def wait(b):
        pltpu.make_async_copy(
            src_hbm.at[pl.ds(0, C)], srows[b], sems[2 * b]).wait()
        pltpu.make_async_copy(
            tgt_hbm.at[pl.ds(0, C)], trows[b], sems[2 * b + 1]).wait()

    def compute(ci, b):
        sb = srows[b]
        tb = trows[b]
        lane = lax.iota(jnp.int32, 16)
        for g in range(C // 16):
            rows = lane + g * 16
            zero = jnp.zeros((16,), jnp.float32)

            def d_blk(k, carry):
                acc0, acc1 = carry
                base = k * UNROLL
                for j in range(UNROLL):
                    col = (jnp.full((16,), base + j, jnp.int32) + lane) & (D - 1)
                    s = plsc.load_gather(sb, [rows, col])
                    t = plsc.load_gather(tb, [rows, col])
                    if j % 2 == 0:
                        acc0 = acc0 + s * t
                    else:
                        acc1 = acc1 + s * t
                return acc0, acc1

            acc0, acc1 = lax.fori_loop(0, D // UNROLL, d_blk, (zero, zero))
            out_v[pl.ds(ci * C + g * 16, 16)] = acc0 + acc1

    for b in range(NBUF):
        fire(b, b)

    def loop_body(i, carry):
        for b in range(NBUF):
            ci = i * NBUF + b

            @pl.when(ci < NCHUNK)
            def _():
                wait(b)
                compute(ci, b)

                @pl.when(ci + NBUF < NCHUNK)
                def _():
                    fire(ci + NBUF, b)

        return carry

    lax.fori_loop(0, (NCHUNK + NBUF - 1) // NBUF, loop_body, 0)
    pltpu.sync_copy(out_v, out_hbm.at[pl.ds(wbase, EPW)])


def kernel(node_src_feats, node_tgt_feats, edge_ids):
    eids = edge_ids.astype(jnp.int32)
    mesh = plsc.VectorSubcoreMesh(core_axis_name="c", subcore_axis_name="s")
    fn = pl.kernel(
        _edge_dot_body,
        out_type=jax.ShapeDtypeStruct((E,), jnp.float32),
        mesh=mesh,
        scratch_types=[
            pltpu.VMEM((EPW,), jnp.int32),
            pltpu.VMEM((EPW,), jnp.int32),
            pltpu.VMEM((EPW,), jnp.float32),
        ] + [pltpu.VMEM((C, D), jnp.float32) for _ in range(2 * NBUF)]
          + [pltpu.SemaphoreType.DMA for _ in range(2 * NBUF)],
        compiler_params=pltpu.CompilerParams(
            needs_layout_passes=False, use_tc_tiling_on_sc=False),
    )
    return fn(node_src_feats, node_tgt_feats, eids)
